# Initial kernel scaffold; baseline (speedup 1.0000x reference)
import jax
import jax.numpy as jnp
from jax.experimental import pallas as pl


def kernel(x, table):
    raise NotImplementedError("write your pallas kernel here")


# speedup vs baseline: 1.5738x; 1.5738x over previous
import functools

import jax
import jax.numpy as jnp
from jax import lax
from jax.experimental import pallas as pl
from jax.experimental.pallas import tpu as pltpu
from jax.experimental.pallas import tpu_sc as plsc

VOCAB = 1000000
EMB = 64
BATCH = 16384
HIST = 50

_B = BATCH * HIST
_NW = 32
_B_PER_W = _B // _NW
_CHUNK = 128
_NSTEPS = _B_PER_W // _CHUNK

_mesh = plsc.VectorSubcoreMesh(core_axis_name="c", subcore_axis_name="s")


@functools.partial(
    pl.kernel,
    out_type=jax.ShapeDtypeStruct((_B, EMB), jnp.float32),
    mesh=_mesh,
    scratch_types=[
        pltpu.VMEM((_CHUNK,), jnp.int32),
        pltpu.VMEM((_CHUNK, EMB), jnp.float32),
        pltpu.SemaphoreType.DMA,
    ],
    compiler_params=pltpu.CompilerParams(use_tc_tiling_on_sc=False),
)
def _gather_kernel(table_hbm, idx_hbm, out_hbm, idx_v, rows_v, sem):
    wid = lax.axis_index("s") * 2 + lax.axis_index("c")
    base = wid * _B_PER_W

    def body(i, _):
        off = base + i * _CHUNK
        pltpu.sync_copy(idx_hbm.at[pl.ds(off, _CHUNK)], idx_v)
        pltpu.async_copy(table_hbm.at[idx_v], rows_v, sem).wait()
        pltpu.sync_copy(rows_v, out_hbm.at[pl.ds(off, _CHUNK)])
        return 0

    lax.fori_loop(0, _NSTEPS, body, 0)


@jax.jit
def kernel(x, table):
    flat_idx = x.reshape(_B).astype(jnp.int32)
    out = _gather_kernel(table, flat_idx)
    return out.reshape(BATCH, HIST, EMB)


# speedup vs baseline: 1.7945x; 1.1403x over previous
import functools

import jax
import jax.numpy as jnp
from jax import lax
from jax.experimental import pallas as pl
from jax.experimental.pallas import tpu as pltpu
from jax.experimental.pallas import tpu_sc as plsc

VOCAB = 1000000
EMB = 64
BATCH = 16384
HIST = 50

_B = BATCH * HIST
_NW = 32
_B_PER_W = _B // _NW
_CHUNK = 512
_NSTEPS = _B_PER_W // _CHUNK

_mesh = plsc.VectorSubcoreMesh(core_axis_name="c", subcore_axis_name="s")


@functools.partial(
    pl.kernel,
    out_type=jax.ShapeDtypeStruct((_B, EMB), jnp.float32),
    mesh=_mesh,
    scratch_types=[
        pltpu.VMEM((_CHUNK,), jnp.int32),
        pltpu.VMEM((_CHUNK, EMB), jnp.float32),
        pltpu.SemaphoreType.DMA,
    ],
    compiler_params=pltpu.CompilerParams(use_tc_tiling_on_sc=False),
)
def _gather_kernel(table_hbm, idx_hbm, out_hbm, idx_v, rows_v, sem):
    wid = lax.axis_index("s") * 2 + lax.axis_index("c")
    base = wid * _B_PER_W

    def body(i, _):
        off = base + i * _CHUNK
        pltpu.sync_copy(idx_hbm.at[pl.ds(off, _CHUNK)], idx_v)
        pltpu.async_copy(table_hbm.at[idx_v], rows_v, sem).wait()
        pltpu.sync_copy(rows_v, out_hbm.at[pl.ds(off, _CHUNK)])
        return 0

    lax.fori_loop(0, _NSTEPS, body, 0)


@jax.jit
def kernel(x, table):
    flat_idx = x.reshape(_B).astype(jnp.int32)
    out = _gather_kernel(table, flat_idx)
    return out.reshape(BATCH, HIST, EMB)


# speedup vs baseline: 1.8524x; 1.0323x over previous
import functools

import jax
import jax.numpy as jnp
from jax import lax
from jax.experimental import pallas as pl
from jax.experimental.pallas import tpu as pltpu
from jax.experimental.pallas import tpu_sc as plsc

VOCAB = 1000000
EMB = 64
BATCH = 16384
HIST = 50

_B = BATCH * HIST
_NW = 32
_B_PER_W = _B // _NW
_CHUNK = 1024
_NSTEPS = _B_PER_W // _CHUNK

_mesh = plsc.VectorSubcoreMesh(core_axis_name="c", subcore_axis_name="s")


@functools.partial(
    pl.kernel,
    out_type=jax.ShapeDtypeStruct((_B, EMB), jnp.float32),
    mesh=_mesh,
    scratch_types=[
        pltpu.VMEM((_CHUNK,), jnp.int32),
        pltpu.VMEM((_CHUNK, EMB), jnp.float32),
        pltpu.SemaphoreType.DMA,
    ],
    compiler_params=pltpu.CompilerParams(use_tc_tiling_on_sc=False),
)
def _gather_kernel(table_hbm, idx_hbm, out_hbm, idx_v, rows_v, sem):
    wid = lax.axis_index("s") * 2 + lax.axis_index("c")
    base = wid * _B_PER_W

    def body(i, _):
        off = base + i * _CHUNK
        pltpu.sync_copy(idx_hbm.at[pl.ds(off, _CHUNK)], idx_v)
        pltpu.async_copy(table_hbm.at[idx_v], rows_v, sem).wait()
        pltpu.sync_copy(rows_v, out_hbm.at[pl.ds(off, _CHUNK)])
        return 0

    lax.fori_loop(0, _NSTEPS, body, 0)


@jax.jit
def kernel(x, table):
    flat_idx = x.reshape(_B).astype(jnp.int32)
    out = _gather_kernel(table, flat_idx)
    return out.reshape(BATCH, HIST, EMB)


# speedup vs baseline: 1.8674x; 1.0081x over previous
import functools

import jax
import jax.numpy as jnp
from jax import lax
from jax.experimental import pallas as pl
from jax.experimental.pallas import tpu as pltpu
from jax.experimental.pallas import tpu_sc as plsc

VOCAB = 1000000
EMB = 64
BATCH = 16384
HIST = 50

_B = BATCH * HIST
_NW = 32
_B_PER_W = _B // _NW
_CHUNK = 512
_NSTEPS = _B_PER_W // _CHUNK

_mesh = plsc.VectorSubcoreMesh(core_axis_name="c", subcore_axis_name="s")


@functools.partial(
    pl.kernel,
    out_type=jax.ShapeDtypeStruct((_B, EMB), jnp.float32),
    mesh=_mesh,
    scratch_types=[
        pltpu.VMEM((_CHUNK,), jnp.int32),
        pltpu.VMEM((_CHUNK,), jnp.int32),
        pltpu.VMEM((_CHUNK, EMB), jnp.float32),
        pltpu.VMEM((_CHUNK, EMB), jnp.float32),
        pltpu.SemaphoreType.DMA,
        pltpu.SemaphoreType.DMA,
        pltpu.SemaphoreType.DMA,
        pltpu.SemaphoreType.DMA,
        pltpu.SemaphoreType.DMA,
        pltpu.SemaphoreType.DMA,
    ],
    compiler_params=pltpu.CompilerParams(use_tc_tiling_on_sc=False),
)
def _gather_kernel(table_hbm, idx_hbm, out_hbm,
                   idx0, idx1, rows0, rows1,
                   si0, si1, sg0, sg1, ss0, ss1):
    wid = lax.axis_index("s") * 2 + lax.axis_index("c")
    base = wid * _B_PER_W

    def idx_load(c, idxv, sem):
        pltpu.async_copy(idx_hbm.at[pl.ds(base + c * _CHUNK, _CHUNK)], idxv, sem)

    def wait_idx(idxv, sem):
        pltpu.make_async_copy(idx_hbm.at[pl.ds(base, _CHUNK)], idxv, sem).wait()

    def store(c, rowsv, sem):
        pltpu.async_copy(rowsv, out_hbm.at[pl.ds(base + c * _CHUNK, _CHUNK)], sem)

    def wait_store(rowsv, sem):
        pltpu.make_async_copy(rowsv, out_hbm.at[pl.ds(base, _CHUNK)], sem).wait()

    idx_load(0, idx0, si0)
    idx_load(1, idx1, si1)

    def body(j, _):
        c0 = 2 * j
        c1 = c0 + 1

        wait_idx(idx0, si0)

        @pl.when(j > 0)
        def _():
            wait_store(rows0, ss0)

        g0 = pltpu.async_copy(table_hbm.at[idx0], rows0, sg0)

        wait_idx(idx1, si1)

        @pl.when(j > 0)
        def _():
            wait_store(rows1, ss1)

        g1 = pltpu.async_copy(table_hbm.at[idx1], rows1, sg1)

        g0.wait()
        store(c0, rows0, ss0)

        @pl.when(c0 + 2 < _NSTEPS)
        def _():
            idx_load(c0 + 2, idx0, si0)

        g1.wait()
        store(c1, rows1, ss1)

        @pl.when(c1 + 2 < _NSTEPS)
        def _():
            idx_load(c1 + 2, idx1, si1)

        return 0

    lax.fori_loop(0, _NSTEPS // 2, body, 0)

    wait_store(rows0, ss0)
    wait_store(rows1, ss1)


@jax.jit
def kernel(x, table):
    flat_idx = x.reshape(_B).astype(jnp.int32)
    out = _gather_kernel(table, flat_idx)
    return out.reshape(BATCH, HIST, EMB)
